# baseline (speedup 1.0000x reference)
import jax
import jax.numpy as jnp
from jax import lax
from jax.experimental import pallas as pl
from jax.experimental.pallas import tpu as pltpu
from jax.experimental.pallas import tpu_sc as plsc

_BATCH = 4096
_HIST = 200
_EMB_DIM = 64
_B = _BATCH * _HIST
_NW = 32
_NROWS = 1000001
_NPAIR = 500004

_mesh = plsc.VectorSubcoreMesh(core_axis_name="c", subcore_axis_name="s")


def _wid():
    return lax.axis_index("s") * 2 + lax.axis_index("c")


def _move_pairs(src, dst, n_pairs):
    def step(i, carry):
        for uu in range(2):
            u = 2 * i + uu
            for k in range(4):
                dst[u, pl.ds(16 * k, 16)] = src[2 * u, pl.ds(16 * k, 16)]
                dst[u, pl.ds(64 + 16 * k, 16)] = (
                    src[2 * u + 1, pl.ds(16 * k, 16)])
        return carry

    lax.fori_loop(0, n_pairs // 2, step, 0)


_D_CHUNK = 192
_D_NFULL = 5208
_D_TAIL0 = _D_NFULL * _D_CHUNK


def _depad_body(table_hbm, tp_hbm, bufa, bufb, buf2a, buf2b,
                rsa, rsb, wsa, wsb):
    wid = _wid()
    base = wid * 162 + jnp.minimum(wid, 24)

    buf = (bufa, bufb)
    buf2 = (buf2a, buf2b)
    rs = (rsa, rsb)
    ws = (wsa, wsb)

    def read(c, i):
        r0 = pl.multiple_of((base + c) * _D_CHUNK, 8)
        return pltpu.make_async_copy(
            table_hbm.at[pl.ds(r0, _D_CHUNK)], buf[i], rs[i])

    def write(c, i):
        p0 = pl.multiple_of((base + c) * (_D_CHUNK // 2), 8)
        return pltpu.make_async_copy(
            buf2[i], tp_hbm.at[pl.ds(p0, _D_CHUNK // 2)], ws[i])

    read(0, 0).start()

    def step(t, carry):
        c0 = 2 * t
        c1 = c0 + 1

        read(c0, 0).wait()
        read(c1, 1).start()

        @pl.when(t > 0)
        def _():
            write(c0 - 2, 0).wait()

        _move_pairs(bufa, buf2a, _D_CHUNK // 2)
        write(c0, 0).start()

        read(c1, 1).wait()

        @pl.when(t < 80)
        def _():
            read(c0 + 2, 0).start()

        @pl.when(t > 0)
        def _():
            write(c1 - 2, 1).wait()

        _move_pairs(bufb, buf2b, _D_CHUNK // 2)
        write(c1, 1).start()
        return carry

    lax.fori_loop(0, 81, step, 0)
    write(160, 0).wait()
    write(161, 1).wait()

    @pl.when(wid < 24)
    def _():
        pltpu.sync_copy(
            table_hbm.at[pl.ds(pl.multiple_of((base + 162) * _D_CHUNK, 8),
                               _D_CHUNK)], bufa)
        _move_pairs(bufa, buf2a, _D_CHUNK // 2)
        pltpu.sync_copy(
            buf2a,
            tp_hbm.at[pl.ds(pl.multiple_of((base + 162) * (_D_CHUNK // 2), 8),
                            _D_CHUNK // 2)])

    @pl.when(wid == 31)
    def _():
        pltpu.sync_copy(table_hbm.at[pl.ds(_D_TAIL0, 64)],
                        bufa.at[pl.ds(0, 64)])
        _move_pairs(bufa, buf2a, 32)
        pltpu.sync_copy(buf2a.at[pl.ds(0, 32)],
                        tp_hbm.at[pl.ds(_D_TAIL0 // 2, 32)])
        pltpu.sync_copy(table_hbm.at[pl.ds(_NROWS - 1, 1)],
                        bufa.at[pl.ds(0, 1)])

        def last(k, carry):
            buf2a[0, pl.ds(16 * k, 16)] = bufa[0, pl.ds(16 * k, 16)]
            return carry

        lax.fori_loop(0, 4, last, 0)
        pltpu.sync_copy(buf2a.at[pl.ds(0, 1)],
                        tp_hbm.at[pl.ds((_NROWS - 1) // 2, 1)])


_depad = pl.kernel(
    _depad_body,
    out_type=jax.ShapeDtypeStruct((_NPAIR, 128), jnp.float32),
    mesh=_mesh,
    scratch_types=[
        pltpu.VMEM((_D_CHUNK, _EMB_DIM), jnp.float32),
        pltpu.VMEM((_D_CHUNK, _EMB_DIM), jnp.float32),
        pltpu.VMEM((_D_CHUNK // 2, 128), jnp.float32),
        pltpu.VMEM((_D_CHUNK // 2, 128), jnp.float32),
        pltpu.SemaphoreType.DMA,
        pltpu.SemaphoreType.DMA,
        pltpu.SemaphoreType.DMA,
        pltpu.SemaphoreType.DMA,
    ],
)


_B_PER_W = _B // _NW
_CHUNK = 800
_N_CHUNKS = _B_PER_W // _CHUNK


def _gather_body(x_hbm, table_hbm, out_hbm, idx_v, rows0, rows1, gsem0, gsem1,
                 osem0, osem1):
    base = _wid() * _B_PER_W

    pltpu.sync_copy(x_hbm.at[pl.ds(base, _B_PER_W)], idx_v)

    rows = (rows0, rows1)
    gsem = (gsem0, gsem1)
    osem = (osem0, osem1)

    def gather(g, buf):
        return pltpu.make_async_copy(
            table_hbm.at[idx_v.at[pl.ds(g * _CHUNK, _CHUNK)]],
            rows[buf], gsem[buf])

    def flush(g, buf):
        return pltpu.make_async_copy(
            rows[buf], out_hbm.at[pl.ds(base + g * _CHUNK, _CHUNK)],
            osem[buf])

    gather(0, 0).start()

    def step(t, carry):
        g0 = 2 * t
        g1 = g0 + 1

        @pl.when(t > 0)
        def _():
            flush(g0 - 1, 1).wait()

        gather(g1, 1).start()
        gather(g0, 0).wait()
        flush(g0, 0).start()

        flush(g0, 0).wait()

        @pl.when(t < _N_CHUNKS // 2 - 1)
        def _():
            gather(g0 + 2, 0).start()

        gather(g1, 1).wait()
        flush(g1, 1).start()
        return carry

    lax.fori_loop(0, _N_CHUNKS // 2, step, 0)
    flush(_N_CHUNKS - 1, 1).wait()


_gather = pl.kernel(
    _gather_body,
    out_type=jax.ShapeDtypeStruct((_B, _EMB_DIM), jnp.float32),
    mesh=_mesh,
    scratch_types=[
        pltpu.VMEM((_B_PER_W,), jnp.int32),
        pltpu.VMEM((_CHUNK, _EMB_DIM), jnp.float32),
        pltpu.VMEM((_CHUNK, _EMB_DIM), jnp.float32),
        pltpu.SemaphoreType.DMA,
        pltpu.SemaphoreType.DMA,
        pltpu.SemaphoreType.DMA,
        pltpu.SemaphoreType.DMA,
    ],
    compiler_params=pltpu.CompilerParams(use_tc_tiling_on_sc=False),
)


_P_CHUNK = 400
_P_NCH = _B_PER_W // _P_CHUNK


def _pad_out_body(emb_hbm, out_hbm, bufca, bufcb, bufd, rsa, rsb):
    wid = _wid()
    base = wid * _B_PER_W

    bufc = (bufca, bufcb)
    rs = (rsa, rsb)

    def read(g, i):
        p0 = pl.multiple_of(base // 2 + g * (_P_CHUNK // 2), 8)
        return pltpu.make_async_copy(
            emb_hbm.at[pl.ds(p0, _P_CHUNK // 2)], bufc[i], rs[i])

    def unpack(i):
        def body(u, carry):
            for b in range(2):
                for k in range(4):
                    bufd[b, 2 * u, pl.ds(16 * k, 16)] = (
                        bufc[i][100 * b + u, pl.ds(16 * k, 16)])
                    bufd[b, 2 * u + 1, pl.ds(16 * k, 16)] = (
                        bufc[i][100 * b + u, pl.ds(64 + 16 * k, 16)])
            return carry

        lax.fori_loop(0, 100, body, 0)

    read(0, 0).start()

    def step(t, carry):
        g0 = 2 * t
        g1 = g0 + 1

        read(g0, 0).wait()
        read(g1, 1).start()
        unpack(0)
        pltpu.sync_copy(bufd, out_hbm.at[pl.ds(wid * 128 + 2 * g0, 2)])

        read(g1, 1).wait()

        @pl.when(t < _P_NCH // 2 - 1)
        def _():
            read(g0 + 2, 0).start()

        unpack(1)
        pltpu.sync_copy(bufd, out_hbm.at[pl.ds(wid * 128 + 2 * g1, 2)])
        return carry

    lax.fori_loop(0, _P_NCH // 2, step, 0)


_pad_out = pl.kernel(
    _pad_out_body,
    out_type=jax.ShapeDtypeStruct((_BATCH, _HIST, _EMB_DIM), jnp.float32),
    mesh=_mesh,
    scratch_types=[
        pltpu.VMEM((_P_CHUNK // 2, 128), jnp.float32),
        pltpu.VMEM((_P_CHUNK // 2, 128), jnp.float32),
        pltpu.VMEM((2, _HIST, _EMB_DIM), jnp.float32),
        pltpu.SemaphoreType.DMA,
        pltpu.SemaphoreType.DMA,
    ],
)


@jax.jit
def kernel(x, table):
    xf = x.reshape(-1).astype(jnp.int32)
    tp = _depad(table)
    emb = _gather(xf, tp.reshape(_NPAIR * 2, _EMB_DIM))
    return _pad_out(emb.reshape(_B // 2, 128))
